# baseline (speedup 1.0000x reference)
import jax
import jax.numpy as jnp
from jax import lax
from jax.experimental import pallas as pl

_BB = 8


def _vq_body(zb_ref, emb_ref, q_ref, idx_ref):
    emb = emb_ref[...]
    n, d = emb.shape
    p = zb_ref.shape[-1]
    e_sq = jnp.sum(emb * emb, axis=1, keepdims=True)
    emb_bf = emb.astype(jnp.bfloat16)
    em2 = emb * -2.0
    iota0 = lax.broadcasted_iota(jnp.int32, (n, p), 0)
    for j in range(_BB):
        zb = zb_ref[j]
        s = lax.dot_general(em2, zb, (((1,), (0,)), ((), ())),
                            preferred_element_type=jnp.float32)
        dist = e_sq + s
        idx = jnp.argmin(dist, axis=0)
        idx_ref[j, 0, :] = idx
        onehot = (iota0 == idx[None, :]).astype(jnp.bfloat16)
        q = lax.dot_general(emb_bf, onehot, (((0,), (0,)), ((), ())),
                            preferred_element_type=jnp.float32)
        q_ref[j] = q


def kernel(z, embedding):
    b, c, h, w = z.shape
    n, d = embedding.shape
    p = h * w
    zr = z.reshape(b, c, p)
    q, idx = pl.pallas_call(
        _vq_body,
        grid=(b // _BB,),
        in_specs=[
            pl.BlockSpec((_BB, c, p), lambda i: (i, 0, 0)),
            pl.BlockSpec((n, d), lambda i: (0, 0)),
        ],
        out_specs=[
            pl.BlockSpec((_BB, c, p), lambda i: (i, 0, 0)),
            pl.BlockSpec((_BB, 1, p), lambda i: (i, 0, 0)),
        ],
        out_shape=[
            jax.ShapeDtypeStruct((b, c, p), jnp.float32),
            jax.ShapeDtypeStruct((b, 1, p), jnp.int32),
        ],
    )(zr, embedding)
    return (q.reshape(b, c, h, w), 0.0, idx.reshape(b, p))
